# baseline (speedup 1.0000x reference)
import functools

import jax
import jax.numpy as jnp
from jax import lax
from jax.experimental import pallas as pl
from jax.experimental.pallas import tpu as pltpu
from jax.experimental.pallas import tpu_sc as plsc

_NBUF = 4


@functools.lru_cache(maxsize=None)
def _build(B, N, V, D, P):
    info = plsc.get_sparse_core_info()
    NC, NS, L = info.num_cores, info.num_subcores, info.num_lanes
    NW = NC * NS
    assert B % NW == 0
    CB = B // NW
    assert CB == 128
    assert D % 8 == 0 and L == 16
    DT = D // 8
    assert V % 2 == 0
    assert N % _NBUF == 0
    PR = (N + 15) // 16 * 8

    mesh = plsc.VectorSubcoreMesh(core_axis_name="c", subcore_axis_name="s")

    @functools.partial(
        pl.kernel,
        mesh=mesh,
        compiler_params=pltpu.CompilerParams(
            use_tc_tiling_on_sc=True, needs_layout_passes=False),
        out_type=jax.ShapeDtypeStruct((N, DT, B // 128, 8, 128), jnp.float32),
        scratch_types=[
            pltpu.VMEM((_NBUF, CB), jnp.int32),
            pltpu.VMEM((_NBUF, CB), jnp.int32),
            pltpu.VMEM((_NBUF, CB, 128), jnp.float32),
            pltpu.VMEM((2, DT, 8, 128), jnp.float32),
            pltpu.VMEM((PR, 128), jnp.float32),
            [pltpu.SemaphoreType.DMA] * _NBUF,
            [pltpu.SemaphoreType.DMA] * _NBUF,
            [pltpu.SemaphoreType.DMA] * 2,
        ],
    )
    def kern(xt_hbm, tab_hbm, pos_hbm, out_hbm, idx_v, jdx_v, buf_v,
             stage_v, pos_v, isems, gsems, ssems):
        wid = lax.axis_index("s") * NC + lax.axis_index("c")
        b0 = wid * CB
        pltpu.sync_copy(pos_hbm.at[pl.ds(0, PR)], pos_v)

        iota = lax.iota(jnp.int32, L)

        def idx_load(n, k):
            return pltpu.async_copy(
                xt_hbm.at[n, pl.ds(b0, CB)], idx_v.at[k], isems[k])

        def idx_wait_and_halve(n, k):
            pltpu.make_async_copy(
                xt_hbm.at[n, pl.ds(b0, CB)], idx_v.at[k], isems[k]).wait()
            for g in range(CB // L):
                sl = pl.ds(g * L, L)
                jdx_v[k, sl] = lax.shift_right_logical(idx_v[k, sl], 1)

        def gather(k):
            return pltpu.async_copy(
                tab_hbm.at[jdx_v.at[k]], buf_v.at[k], gsems[k])

        for n in range(_NBUF):
            idx_load(n, n)
        for n in range(2):
            idx_wait_and_halve(n, n)
            gather(n)

        def outer(j, carry):
            for k in range(_NBUF):
                n = j * _NBUF + k
                k2 = (k + 2) % _NBUF
                sk = k % 2
                pltpu.make_async_copy(
                    tab_hbm.at[jdx_v.at[k]], buf_v.at[k], gsems[k]).wait()
                @pl.when(n + 2 < N)
                def _():
                    idx_wait_and_halve(n + 2, k2)
                    gather(k2)
                @pl.when(n >= 2)
                def _():
                    pltpu.make_async_copy(
                        stage_v.at[sk], out_hbm.at[n - 2, :, wid],
                        ssems[sk]).wait()
                par64 = [
                    lax.shift_left(
                        lax.bitwise_and(idx_v[k, pl.ds(g * L, L)], 1), 6)
                    for g in range(CB // L)
                ]
                rvecs = [iota + g * L for g in range(CB // L)]
                zeros = iota * 0
                prow_vec = zeros + (j * (_NBUF // 2) + k // 2)
                pcol = (k % 2) * D

                @plsc.parallel_loop(0, L, unroll=2)
                def qrow(q):
                    rot = lax.rem(iota + q, L)
                    for c in range(D // L):
                        dvec = rot + c * L
                        dtvec = lax.shift_right_logical(dvec, 3)
                        drvec = lax.bitwise_and(dvec, 7)
                        pvec = plsc.load_gather(pos_v, [prow_vec, dvec + pcol])
                        for g in range(CB // L):
                            val = plsc.load_gather(
                                buf_v.at[k], [rvecs[g], par64[g] + dvec])
                            plsc.store_scatter(
                                stage_v.at[sk], [dtvec, drvec, rvecs[g]],
                                val + pvec)
                pltpu.async_copy(
                    stage_v.at[sk], out_hbm.at[n, :, wid], ssems[sk])

                @pl.when(n + _NBUF < N)
                def _():
                    idx_load(n + _NBUF, k)
            return carry
        lax.fori_loop(0, N // _NBUF, outer, 0)
        for n in range(N - 2, N):
            pltpu.make_async_copy(
                stage_v.at[n % 2], out_hbm.at[n, :, wid],
                ssems[n % 2]).wait()

    return kern


@functools.lru_cache(maxsize=None)
def _build_transpose(V, D):
    info = plsc.get_sparse_core_info()
    NC, NS, L = info.num_cores, info.num_subcores, info.num_lanes
    NW = NC * NS
    assert D == 64 and L == 16
    NCH = V // 128
    TAIL = V - NCH * 128
    TPW = NCH // NW
    assert TPW % 4 == 0 and TPW >= 8

    mesh = plsc.VectorSubcoreMesh(core_axis_name="c", subcore_axis_name="s")

    @functools.partial(
        pl.kernel,
        mesh=mesh,
        compiler_params=pltpu.CompilerParams(
            use_tc_tiling_on_sc=True, needs_layout_passes=False),
        out_type=jax.ShapeDtypeStruct((V // 2, 2 * D), jnp.float32),
        scratch_types=[
            pltpu.VMEM((4, D, 128), jnp.float32),
            pltpu.VMEM((4, D, 2 * D), jnp.float32),
            pltpu.VMEM((D, TAIL), jnp.float32),
            [pltpu.SemaphoreType.DMA] * 4,
            [pltpu.SemaphoreType.DMA] * 4,
        ],
    )
    def kern(tabT_hbm, tail_hbm, out_hbm, in_v, out_v, tail_v, isems, osems):
        wid = lax.axis_index("s") * NC + lax.axis_index("c")
        iota = lax.iota(jnp.int32, L)
        iota4 = iota * 4

        def in_copy(c, r):
            return pltpu.async_copy(
                tabT_hbm.at[:, pl.ds(c * 128, 128)], in_v.at[r], isems[r])

        def transpose_chunk(r):
            @plsc.parallel_loop(0, L, unroll=2)
            def arow(a):
                rot = lax.bitwise_and(iota + a, L - 1)
                for jb in range(4):
                    jpv = iota4 + jb
                    vbase = 2 * jpv
                    for s in range(8):
                        qv = rot * 8 + s
                        dv = lax.bitwise_and(qv, D - 1)
                        vlv = vbase + lax.shift_right_logical(qv, 6)
                        val = plsc.load_gather(in_v.at[r], [dv, vlv])
                        plsc.store_scatter(out_v.at[r], [jpv, qv], val)

        for p in range(3):
            in_copy(wid + NW * p, p)

        def step(t4, carry):
            for rr in range(4):
                t = t4 * 4 + rr
                c = wid + NW * t
                pltpu.make_async_copy(
                    tabT_hbm.at[:, pl.ds(c * 128, 128)],
                    in_v.at[rr], isems[rr]).wait()

                @pl.when(t + 3 < TPW)
                def _():
                    in_copy(c + 3 * NW, (rr + 3) % 4)

                @pl.when(t >= 4)
                def _():
                    pltpu.make_async_copy(
                        out_v.at[rr], out_hbm.at[pl.ds((c - 4 * NW) * D, D)],
                        osems[rr]).wait()
                transpose_chunk(rr)
                pltpu.async_copy(
                    out_v.at[rr], out_hbm.at[pl.ds(c * D, D)], osems[rr])
            return carry
        lax.fori_loop(0, TPW // 4, step, 0)
        for n in range(4):
            pltpu.make_async_copy(
                out_v.at[(TPW - 4 + n) % 4],
                out_hbm.at[pl.ds((wid + (TPW - 4 + n) * NW) * D, D)],
                osems[(TPW - 4 + n) % 4]).wait()

        REM = NCH - TPW * NW

        @pl.when(wid < REM)
        def _():
            c = NW * TPW + wid
            pltpu.sync_copy(tabT_hbm.at[:, pl.ds(c * 128, 128)], in_v.at[0])
            transpose_chunk(0)
            pltpu.sync_copy(out_v.at[0], out_hbm.at[pl.ds(c * D, D)])

        if TAIL:
            @pl.when(wid == REM)
            def _():
                pltpu.sync_copy(tail_hbm, tail_v)

                @plsc.parallel_loop(0, TAIL // 2, unroll=2)
                def jrow3(jp):
                    for cc in range(2 * D // L):
                        rvec = (cc % (D // L)) * L + iota
                        cvec = iota * 0 + (2 * jp + (cc // (D // L)))
                        val = plsc.load_gather(tail_v, [rvec, cvec])
                        out_v[0, jp, pl.ds(cc * L, L)] = val
                pltpu.sync_copy(
                    out_v.at[0, pl.ds(0, TAIL // 2)],
                    out_hbm.at[pl.ds(NCH * D, TAIL // 2)])

    return kern


def kernel(x, emb_table, pos_table):
    B, N = x.shape
    V, D = emb_table.shape
    P = pos_table.shape[0]
    kern = _build(B, N, V, D, P)
    xt = jnp.swapaxes(x, 0, 1).astype(jnp.int32)
    tail = jnp.swapaxes(emb_table[(V // 128) * 128:], 0, 1)
    tab2 = _build_transpose(V, D)(jnp.swapaxes(emb_table, 0, 1), tail)
    pos2 = pos_table.reshape(P // 2, 2 * D)
    out5 = kern(xt, tab2, pos2)
    return jnp.transpose(out5, (2, 4, 0, 1, 3)).reshape(B, N, D)


# speedup vs baseline: 1.0547x; 1.0547x over previous
import functools

import jax
import jax.numpy as jnp
from jax import lax
from jax.experimental import pallas as pl
from jax.experimental.pallas import tpu as pltpu
from jax.experimental.pallas import tpu_sc as plsc

_NBUF = 4


@functools.lru_cache(maxsize=None)
def _build(B, N, V, D, P):
    info = plsc.get_sparse_core_info()
    NC, NS, L = info.num_cores, info.num_subcores, info.num_lanes
    NW = NC * NS
    assert B % NW == 0
    CB = B // NW
    assert CB == 128
    assert D % 8 == 0 and L == 16
    DT = D // 8
    assert V % 2 == 0
    assert N % _NBUF == 0
    PR = (N + 15) // 16 * 8

    mesh = plsc.VectorSubcoreMesh(core_axis_name="c", subcore_axis_name="s")

    @functools.partial(
        pl.kernel,
        mesh=mesh,
        compiler_params=pltpu.CompilerParams(
            use_tc_tiling_on_sc=True, needs_layout_passes=False),
        out_type=jax.ShapeDtypeStruct((N, DT, B // 128, 8, 128), jnp.float32),
        scratch_types=[
            pltpu.VMEM((_NBUF, CB), jnp.int32),
            pltpu.VMEM((_NBUF, CB), jnp.int32),
            pltpu.VMEM((_NBUF, CB, 128), jnp.float32),
            pltpu.VMEM((2, DT, 8, 128), jnp.float32),
            pltpu.VMEM((PR, 128), jnp.float32),
            [pltpu.SemaphoreType.DMA] * _NBUF,
            [pltpu.SemaphoreType.DMA] * _NBUF,
            [pltpu.SemaphoreType.DMA] * 2,
        ],
    )
    def kern(xt_hbm, tab_hbm, pos_hbm, out_hbm, idx_v, jdx_v, buf_v,
             stage_v, pos_v, isems, gsems, ssems):
        wid = lax.axis_index("s") * NC + lax.axis_index("c")
        b0 = wid * CB
        pltpu.sync_copy(pos_hbm.at[pl.ds(0, PR)], pos_v)

        iota = lax.iota(jnp.int32, L)

        def idx_load(n, k):
            return pltpu.async_copy(
                xt_hbm.at[n, pl.ds(b0, CB)], idx_v.at[k], isems[k])

        def idx_wait_and_halve(n, k):
            pltpu.make_async_copy(
                xt_hbm.at[n, pl.ds(b0, CB)], idx_v.at[k], isems[k]).wait()
            for g in range(CB // L):
                sl = pl.ds(g * L, L)
                jdx_v[k, sl] = lax.shift_right_logical(idx_v[k, sl], 1)

        def gather(k):
            return pltpu.async_copy(
                tab_hbm.at[jdx_v.at[k]], buf_v.at[k], gsems[k])

        for n in range(_NBUF):
            idx_load(n, n)
        for n in range(2):
            idx_wait_and_halve(n, n)
            gather(n)

        def outer(j, carry):
            for k in range(_NBUF):
                n = j * _NBUF + k
                k2 = (k + 2) % _NBUF
                sk = k % 2
                pltpu.make_async_copy(
                    tab_hbm.at[jdx_v.at[k]], buf_v.at[k], gsems[k]).wait()
                @pl.when(n + 2 < N)
                def _():
                    idx_wait_and_halve(n + 2, k2)
                    gather(k2)
                @pl.when(n >= 2)
                def _():
                    pltpu.make_async_copy(
                        stage_v.at[sk], out_hbm.at[n - 2, :, wid],
                        ssems[sk]).wait()
                par64 = [
                    lax.shift_left(
                        lax.bitwise_and(idx_v[k, pl.ds(g * L, L)], 1), 6)
                    for g in range(CB // L)
                ]
                rvecs = [iota + g * L for g in range(CB // L)]
                zeros = iota * 0
                prow_vec = zeros + (j * (_NBUF // 2) + k // 2)
                pcol = (k % 2) * D

                @plsc.parallel_loop(0, L, unroll=2)
                def qrow(q):
                    rot = lax.rem(iota + q, L)
                    for c in range(D // L):
                        dvec = rot + c * L
                        dtvec = lax.shift_right_logical(dvec, 3)
                        drvec = lax.bitwise_and(dvec, 7)
                        pvec = plsc.load_gather(pos_v, [prow_vec, dvec + pcol])
                        for g in range(CB // L):
                            val = plsc.load_gather(
                                buf_v.at[k], [rvecs[g], par64[g] + dvec])
                            plsc.store_scatter(
                                stage_v.at[sk], [dtvec, drvec, rvecs[g]],
                                val + pvec)
                pltpu.async_copy(
                    stage_v.at[sk], out_hbm.at[n, :, wid], ssems[sk])

                @pl.when(n + _NBUF < N)
                def _():
                    idx_load(n + _NBUF, k)
            return carry
        lax.fori_loop(0, N // _NBUF, outer, 0)
        for n in range(N - 2, N):
            pltpu.make_async_copy(
                stage_v.at[n % 2], out_hbm.at[n, :, wid],
                ssems[n % 2]).wait()

    return kern


@functools.lru_cache(maxsize=None)
def _build_transpose(V, D):
    info = plsc.get_sparse_core_info()
    NC, NS, L = info.num_cores, info.num_subcores, info.num_lanes
    NW = NC * NS
    assert D == 64 and L == 16
    NCH = V // 128
    TAIL = V - NCH * 128
    TPW = NCH // NW
    assert TPW % 2 == 0

    mesh = plsc.VectorSubcoreMesh(core_axis_name="c", subcore_axis_name="s")

    @functools.partial(
        pl.kernel,
        mesh=mesh,
        compiler_params=pltpu.CompilerParams(
            use_tc_tiling_on_sc=True, needs_layout_passes=False),
        out_type=jax.ShapeDtypeStruct((V // 2, 2 * D), jnp.float32),
        scratch_types=[
            pltpu.VMEM((2, D, 128), jnp.float32),
            pltpu.VMEM((2, D, 2 * D), jnp.float32),
            pltpu.VMEM((D, TAIL), jnp.float32),
            [pltpu.SemaphoreType.DMA] * 2,
            [pltpu.SemaphoreType.DMA] * 2,
        ],
    )
    def kern(tabT_hbm, tail_hbm, out_hbm, in_v, out_v, tail_v, isems, osems):
        wid = lax.axis_index("s") * NC + lax.axis_index("c")
        iota = lax.iota(jnp.int32, L)
        iota4 = iota * 4

        def in_copy(c, r):
            return pltpu.async_copy(
                tabT_hbm.at[:, pl.ds(c * 128, 128)], in_v.at[r], isems[r])

        def transpose_chunk(r):
            @plsc.parallel_loop(0, L, unroll=2)
            def arow(a):
                rot = lax.bitwise_and(iota + a, L - 1)
                for jb in range(4):
                    jpv = iota4 + jb
                    vbase = 2 * jpv
                    for s in range(8):
                        qv = rot * 8 + s
                        dv = lax.bitwise_and(qv, D - 1)
                        vlv = vbase + lax.shift_right_logical(qv, 6)
                        val = plsc.load_gather(in_v.at[r], [dv, vlv])
                        plsc.store_scatter(out_v.at[r], [jpv, qv], val)

        in_copy(wid, 0)

        def step(t2, carry):
            for rr in range(2):
                t = t2 * 2 + rr
                c = wid + NW * t
                pltpu.make_async_copy(
                    tabT_hbm.at[:, pl.ds(c * 128, 128)],
                    in_v.at[rr], isems[rr]).wait()

                @pl.when(t + 1 < TPW)
                def _():
                    in_copy(c + NW, 1 - rr)

                @pl.when(t >= 2)
                def _():
                    pltpu.make_async_copy(
                        out_v.at[rr], out_hbm.at[pl.ds((c - 2 * NW) * D, D)],
                        osems[rr]).wait()
                transpose_chunk(rr)
                pltpu.async_copy(
                    out_v.at[rr], out_hbm.at[pl.ds(c * D, D)], osems[rr])
            return carry
        lax.fori_loop(0, TPW // 2, step, 0)
        for n in range(2):
            pltpu.make_async_copy(
                out_v.at[n % 2],
                out_hbm.at[pl.ds((wid + (TPW - 2 + n) * NW) * D, D)],
                osems[n % 2]).wait()

        REM = NCH - TPW * NW

        @pl.when(wid < REM)
        def _():
            c = NW * TPW + wid
            pltpu.sync_copy(tabT_hbm.at[:, pl.ds(c * 128, 128)], in_v.at[0])
            transpose_chunk(0)
            pltpu.sync_copy(out_v.at[0], out_hbm.at[pl.ds(c * D, D)])

        if TAIL:
            @pl.when(wid == REM)
            def _():
                pltpu.sync_copy(tail_hbm, tail_v)

                @plsc.parallel_loop(0, TAIL // 2, unroll=2)
                def jrow3(jp):
                    for cc in range(2 * D // L):
                        rvec = (cc % (D // L)) * L + iota
                        cvec = iota * 0 + (2 * jp + (cc // (D // L)))
                        val = plsc.load_gather(tail_v, [rvec, cvec])
                        out_v[0, jp, pl.ds(cc * L, L)] = val
                pltpu.sync_copy(
                    out_v.at[0, pl.ds(0, TAIL // 2)],
                    out_hbm.at[pl.ds(NCH * D, TAIL // 2)])

    return kern


def kernel(x, emb_table, pos_table):
    B, N = x.shape
    V, D = emb_table.shape
    P = pos_table.shape[0]
    kern = _build(B, N, V, D, P)
    xt = jnp.swapaxes(x, 0, 1).astype(jnp.int32)
    tail = jnp.swapaxes(emb_table[(V // 128) * 128:], 0, 1)
    tab2 = _build_transpose(V, D)(jnp.swapaxes(emb_table, 0, 1), tail)
    pos2 = pos_table.reshape(P // 2, 2 * D)
    out5 = kern(xt, tab2, pos2)
    return jnp.transpose(out5, (2, 4, 0, 1, 3)).reshape(B, N, D)
